# baseline (speedup 1.0000x reference)
import functools

import jax
import jax.numpy as jnp
from jax import lax
from jax.experimental import pallas as pl
from jax.experimental.pallas import tpu as pltpu
from jax.experimental.pallas import tpu_sc as plsc

UNITS = 26
LOWER = -17.0
UPPER = 8.0
BW = 0.05
NUM_BUCKETS = int((UPPER - LOWER) / BW) + 1
RESIDUE = LOWER - BW
BATCH = 4096

NC = 2
NS = 16
NW = NC * NS
LANES = 16

ROWS_PER_TILE = BATCH // NW
GROUPS = ROWS_PER_TILE // LANES

assert ROWS_PER_TILE * NW == BATCH and GROUPS * LANES == ROWS_PER_TILE


KCOLS = NUM_BUCKETS + 1


def _tables_body(w_ref, b_ref, a_ref):
    r = jnp.maximum(w_ref[...], jnp.float32(0.0))
    ki = lax.broadcasted_iota(jnp.int32, (NUM_BUCKETS, KCOLS), 0)
    ji = lax.broadcasted_iota(jnp.int32, (NUM_BUCKETS, KCOLS), 1)
    tri = jnp.where(ki < ji, jnp.float32(BW), jnp.float32(0.0))
    acc = lax.dot(r, tri, precision=lax.Precision.HIGHEST,
                  preferred_element_type=jnp.float32)
    ones_row = jnp.full((1, KCOLS), 1.0, jnp.float32)
    bb = lax.dot_general(b_ref[...], ones_row,
                         dimension_numbers=(((0,), (0,)), ((), ())),
                         precision=lax.Precision.HIGHEST,
                         preferred_element_type=jnp.float32)
    a_ref[...] = acc + bb + jnp.float32(RESIDUE)


def _sc_body(x_hbm, a_hbm, o_hbm, x_v, a_v, c_v, o_v, sem):
    wid = lax.axis_index("s") * NC + lax.axis_index("c")
    col0 = wid * ROWS_PER_TILE
    cx = pltpu.async_copy(x_hbm.at[:, pl.ds(col0, ROWS_PER_TILE)], x_v, sem)
    ca = pltpu.async_copy(a_hbm, a_v, sem)
    cx.wait()

    clo = jnp.float32(LOWER + 1e-09)
    chi = jnp.float32(UPPER - 1e-09)
    shiftf = jnp.float32(BW - LOWER)
    invbw = jnp.float32(1.0 / BW)
    idxcap = jnp.float32(NUM_BUCKETS - 0.5)
    half = jnp.float32(0.5)
    two = jnp.float32(2.0)
    one = jnp.float32(1.0)

    @plsc.parallel_loop(0, UNITS * GROUPS, unroll=16)
    def bucketize(i):
        u = i >> 3
        off = (i & (GROUPS - 1)) * LANES
        xc = jnp.clip(x_v.at[u][pl.ds(off, LANES)], clo, chi)
        t = (xc + shiftf) * invbw
        idxf = jnp.minimum(t, idxcap).astype(jnp.int32).astype(jnp.float32)
        c_v.at[u][pl.ds(off, LANES)] = idxf + (t - idxf) * half

    ca.wait()

    @plsc.parallel_loop(0, UNITS * GROUPS, unroll=16)
    def combine(i):
        u = i >> 3
        off = (i & (GROUPS - 1)) * LANES
        ucol = jnp.full((LANES,), u, jnp.int32)
        c = c_v.at[u][pl.ds(off, LANES)]
        idx = c.astype(jnp.int32)
        frac2 = c - idx.astype(jnp.float32)
        a0 = plsc.load_gather(a_v, [ucol, idx])
        a1 = plsc.load_gather(a_v, [ucol, idx + 1])
        z = a0 + frac2 * two * (a1 - a0)
        o_v.at[u][pl.ds(off, LANES)] = one / (one + jnp.exp(-z))

    pltpu.sync_copy(o_v, o_hbm.at[:, pl.ds(col0, ROWS_PER_TILE)])


def kernel(x, weights, bias):
    a2 = pl.pallas_call(
        _tables_body,
        out_shape=jax.ShapeDtypeStruct((UNITS, KCOLS), jnp.float32),
        compiler_params=pltpu.CompilerParams(skip_device_barrier=True),
    )(weights.astype(jnp.float32), bias.astype(jnp.float32)[None, :])

    sc = functools.partial(
        pl.kernel,
        out_type=jax.ShapeDtypeStruct((UNITS, BATCH), jnp.float32),
        mesh=plsc.VectorSubcoreMesh(core_axis_name="c", subcore_axis_name="s"),
        scratch_types=[
            pltpu.VMEM((UNITS, ROWS_PER_TILE), jnp.float32),
            pltpu.VMEM((UNITS, KCOLS), jnp.float32),
            pltpu.VMEM((UNITS, ROWS_PER_TILE), jnp.float32),
            pltpu.VMEM((UNITS, ROWS_PER_TILE), jnp.float32),
            pltpu.SemaphoreType.DMA,
        ],
        compiler_params=pltpu.CompilerParams(needs_layout_passes=False,
                                             use_tc_tiling_on_sc=True,
                                             skip_device_barrier=True),
    )(_sc_body)
    return sc(x.T, a2).T


# speedup vs baseline: 1.0287x; 1.0287x over previous
import functools

import jax
import jax.numpy as jnp
from jax import lax
from jax.experimental import pallas as pl
from jax.experimental.pallas import tpu as pltpu
from jax.experimental.pallas import tpu_sc as plsc

UNITS = 26
LOWER = -17.0
UPPER = 8.0
BW = 0.05
NUM_BUCKETS = int((UPPER - LOWER) / BW) + 1
RESIDUE = LOWER - BW
BATCH = 4096

NC = 2
NS = 16
NW = NC * NS
LANES = 16

ROWS_PER_TILE = BATCH // NW
GROUPS = ROWS_PER_TILE // LANES

assert ROWS_PER_TILE * NW == BATCH and GROUPS * LANES == ROWS_PER_TILE


KCOLS = NUM_BUCKETS + 1


def _tables_body(w_ref, b_ref, a_ref):
    r = jnp.maximum(w_ref[...], jnp.float32(0.0))
    ki = lax.broadcasted_iota(jnp.int32, (NUM_BUCKETS, KCOLS), 0)
    ji = lax.broadcasted_iota(jnp.int32, (NUM_BUCKETS, KCOLS), 1)
    tri = jnp.where(ki < ji, jnp.float32(BW), jnp.float32(0.0))
    acc = lax.dot(r, tri, precision=lax.Precision.HIGHEST,
                  preferred_element_type=jnp.float32)
    ones_row = jnp.full((1, KCOLS), 1.0, jnp.float32)
    bb = lax.dot_general(b_ref[...], ones_row,
                         dimension_numbers=(((0,), (0,)), ((), ())),
                         precision=lax.Precision.HIGHEST,
                         preferred_element_type=jnp.float32)
    a_ref[...] = acc + bb + jnp.float32(RESIDUE)


def _sc_body(x_hbm, a_hbm, o_hbm, x_v, a_v, c_v, o_v, sem):
    wid = lax.axis_index("s") * NC + lax.axis_index("c")
    col0 = wid * ROWS_PER_TILE
    ca = pltpu.async_copy(a_hbm, a_v, sem)
    cx = pltpu.async_copy(x_hbm.at[:, pl.ds(col0, ROWS_PER_TILE)], x_v, sem)
    cx.wait()

    clo = jnp.float32(LOWER + 1e-09)
    chi = jnp.float32(UPPER - 1e-09)
    shiftf = jnp.float32(BW - LOWER)
    invbw = jnp.float32(1.0 / BW)
    idxcap = jnp.float32(NUM_BUCKETS - 0.5)
    half = jnp.float32(0.5)
    two = jnp.float32(2.0)
    one = jnp.float32(1.0)

    @plsc.parallel_loop(0, UNITS * GROUPS, unroll=8)
    def bucketize(i):
        u = i >> 3
        off = (i & (GROUPS - 1)) * LANES
        xc = jnp.clip(x_v.at[u][pl.ds(off, LANES)], clo, chi)
        t = (xc + shiftf) * invbw
        idxf = jnp.minimum(t, idxcap).astype(jnp.int32).astype(jnp.float32)
        c_v.at[u][pl.ds(off, LANES)] = idxf + (t - idxf) * half

    ca.wait()

    @plsc.parallel_loop(0, UNITS * GROUPS, unroll=8)
    def combine(i):
        u = i >> 3
        off = (i & (GROUPS - 1)) * LANES
        ucol = jnp.full((LANES,), u, jnp.int32)
        c = c_v.at[u][pl.ds(off, LANES)]
        idx = c.astype(jnp.int32)
        frac2 = c - idx.astype(jnp.float32)
        a0 = plsc.load_gather(a_v, [ucol, idx])
        a1 = plsc.load_gather(a_v, [ucol, idx + 1])
        z = a0 + frac2 * two * (a1 - a0)
        o_v.at[u][pl.ds(off, LANES)] = one / (one + jnp.exp(-z))

    pltpu.sync_copy(o_v, o_hbm.at[:, pl.ds(col0, ROWS_PER_TILE)])


def kernel(x, weights, bias):
    a2 = pl.pallas_call(
        _tables_body,
        out_shape=jax.ShapeDtypeStruct((UNITS, KCOLS), jnp.float32),
        compiler_params=pltpu.CompilerParams(skip_device_barrier=True),
    )(weights.astype(jnp.float32), bias.astype(jnp.float32)[None, :])

    sc = functools.partial(
        pl.kernel,
        out_type=jax.ShapeDtypeStruct((UNITS, BATCH), jnp.float32),
        mesh=plsc.VectorSubcoreMesh(core_axis_name="c", subcore_axis_name="s"),
        scratch_types=[
            pltpu.VMEM((UNITS, ROWS_PER_TILE), jnp.float32),
            pltpu.VMEM((UNITS, KCOLS), jnp.float32),
            pltpu.VMEM((UNITS, ROWS_PER_TILE), jnp.float32),
            pltpu.VMEM((UNITS, ROWS_PER_TILE), jnp.float32),
            pltpu.SemaphoreType.DMA,
        ],
        compiler_params=pltpu.CompilerParams(needs_layout_passes=False,
                                             use_tc_tiling_on_sc=True,
                                             skip_device_barrier=True),
    )(_sc_body)
    return sc(x.T, a2).T


# speedup vs baseline: 1.0307x; 1.0020x over previous
import functools

import jax
import jax.numpy as jnp
from jax import lax
from jax.experimental import pallas as pl
from jax.experimental.pallas import tpu as pltpu
from jax.experimental.pallas import tpu_sc as plsc

UNITS = 26
LOWER = -17.0
UPPER = 8.0
BW = 0.05
NUM_BUCKETS = int((UPPER - LOWER) / BW) + 1
RESIDUE = LOWER - BW
BATCH = 4096

NC = 2
NS = 16
NW = NC * NS
LANES = 16

ROWS_PER_TILE = BATCH // NW
GROUPS = ROWS_PER_TILE // LANES

assert ROWS_PER_TILE * NW == BATCH and GROUPS * LANES == ROWS_PER_TILE


KCOLS = NUM_BUCKETS + 1


def _tables_body(w_ref, b_ref, a_ref):
    r = jnp.maximum(w_ref[...], jnp.float32(0.0))
    ki = lax.broadcasted_iota(jnp.int32, (NUM_BUCKETS, KCOLS), 0)
    ji = lax.broadcasted_iota(jnp.int32, (NUM_BUCKETS, KCOLS), 1)
    tri = jnp.where(ki < ji, jnp.float32(BW), jnp.float32(0.0))
    acc = lax.dot(r, tri, precision=lax.Precision.HIGHEST,
                  preferred_element_type=jnp.float32)
    ones_row = jnp.full((1, KCOLS), 1.0, jnp.float32)
    bb = lax.dot_general(b_ref[...], ones_row,
                         dimension_numbers=(((0,), (0,)), ((), ())),
                         precision=lax.Precision.HIGHEST,
                         preferred_element_type=jnp.float32)
    a_ref[...] = acc + bb + jnp.float32(RESIDUE)


def _sc_body(x_hbm, a_hbm, o_hbm, x_v, a_v, c_v, o_v, sem):
    wid = lax.axis_index("s") * NC + lax.axis_index("c")
    col0 = wid * ROWS_PER_TILE
    cx = pltpu.async_copy(x_hbm.at[:, pl.ds(col0, ROWS_PER_TILE)], x_v, sem)
    ca = pltpu.async_copy(a_hbm, a_v, sem)
    cx.wait()

    clo = jnp.float32(LOWER + 1e-09)
    chi = jnp.float32(UPPER - 1e-09)
    shiftf = jnp.float32(BW - LOWER)
    invbw = jnp.float32(1.0 / BW)
    idxcap = jnp.float32(NUM_BUCKETS - 0.5)
    half = jnp.float32(0.5)
    two = jnp.float32(2.0)
    one = jnp.float32(1.0)

    @plsc.parallel_loop(0, UNITS * GROUPS, unroll=8)
    def bucketize(i):
        u = i >> 3
        off = (i & (GROUPS - 1)) * LANES
        xc = jnp.clip(x_v.at[u][pl.ds(off, LANES)], clo, chi)
        t = (xc + shiftf) * invbw
        idxf = jnp.minimum(t, idxcap).astype(jnp.int32).astype(jnp.float32)
        c_v.at[u][pl.ds(off, LANES)] = idxf + (t - idxf) * half

    ca.wait()

    @plsc.parallel_loop(0, UNITS * GROUPS, unroll=8)
    def combine(i):
        u = i >> 3
        off = (i & (GROUPS - 1)) * LANES
        ucol = jnp.full((LANES,), u, jnp.int32)
        c = c_v.at[u][pl.ds(off, LANES)]
        idx = c.astype(jnp.int32)
        frac2 = c - idx.astype(jnp.float32)
        a0 = plsc.load_gather(a_v, [ucol, idx])
        a1 = plsc.load_gather(a_v, [ucol, idx + 1])
        z = a0 + frac2 * two * (a1 - a0)
        o_v.at[u][pl.ds(off, LANES)] = one / (one + jnp.exp(-z))

    pltpu.sync_copy(o_v, o_hbm.at[:, pl.ds(col0, ROWS_PER_TILE)])


def kernel(x, weights, bias):
    a2 = pl.pallas_call(
        _tables_body,
        out_shape=jax.ShapeDtypeStruct((UNITS, KCOLS), jnp.float32),
        compiler_params=pltpu.CompilerParams(skip_device_barrier=True),
    )(weights.astype(jnp.float32), bias.astype(jnp.float32)[None, :])

    sc = functools.partial(
        pl.kernel,
        out_type=jax.ShapeDtypeStruct((UNITS, BATCH), jnp.float32),
        mesh=plsc.VectorSubcoreMesh(core_axis_name="c", subcore_axis_name="s"),
        scratch_types=[
            pltpu.VMEM((UNITS, ROWS_PER_TILE), jnp.float32),
            pltpu.VMEM((UNITS, KCOLS), jnp.float32),
            pltpu.VMEM((UNITS, ROWS_PER_TILE), jnp.float32),
            pltpu.VMEM((UNITS, ROWS_PER_TILE), jnp.float32),
            pltpu.SemaphoreType.DMA,
        ],
        compiler_params=pltpu.CompilerParams(needs_layout_passes=False,
                                             use_tc_tiling_on_sc=True,
                                             skip_device_barrier=True),
    )(_sc_body)
    return sc(x.T, a2).T
